# baseline (speedup 1.0000x reference)
import functools

import jax
import jax.numpy as jnp
from jax import lax
from jax.experimental import pallas as pl
from jax.experimental.pallas import tpu as pltpu
from jax.experimental.pallas import tpu_sc as plsc

NUM_EMB = 1_000_000
DIM = 32
BATCH = 16384
FIELDS = 100
TOTAL = BATCH * FIELDS

NUM_CORES = 2
NUM_SUBCORES = 16
NUM_WORKERS = NUM_CORES * NUM_SUBCORES
CHUNK = BATCH // NUM_WORKERS
NSLOT = 2
LANES = 16

WCH = 512
N_WCH = NUM_EMB // WCH - 1
N_WT = 62
XF = 8
XB = 2048
X_TAIL_F = 96


def _build_prep():
    mesh = plsc.VectorSubcoreMesh(core_axis_name="c", subcore_axis_name="s")

    @functools.partial(
        pl.kernel,
        mesh=mesh,
        out_type=(
            jax.ShapeDtypeStruct((NUM_EMB * DIM,), jnp.float32),
            jax.ShapeDtypeStruct((TOTAL,), jnp.int32),
        ),
        scratch_types=[
            [pltpu.VMEM((DIM, WCH), jnp.float32) for _ in range(2)],
            [pltpu.VMEM((WCH * DIM,), jnp.float32) for _ in range(2)],
            pltpu.VMEM((XF, XB), jnp.int32),
            pltpu.VMEM((XF * XB,), jnp.int32),
            [pltpu.SemaphoreType.DMA for _ in range(2)],
            [pltpu.SemaphoreType.DMA for _ in range(2)],
        ],
        compiler_params=pltpu.CompilerParams(
            use_tc_tiling_on_sc=True, needs_layout_passes=False),
    )
    def prep_kernel(wt_hbm, xt_hbm, wtail_hbm, xtail_hbm, wflat_hbm,
                    idx_hbm, win_v, wout_v, xbuf_v, xstage_v, win_sem,
                    wout_sem):
        wid = lax.axis_index("s") * NUM_CORES + lax.axis_index("c")
        lane_iota = lax.iota(jnp.int32, LANES)
        scat_base = lane_iota * DIM

        for k in range(3):
            xb = wid * 3 + k
            f0 = (xb // (BATCH // XB)) * XF
            b0 = (xb % (BATCH // XB)) * XB
            pltpu.sync_copy(
                xt_hbm.at[pl.ds(f0, XF), pl.ds(b0, XB)], xbuf_v)

            def cg_body(cg, carry):
                c0 = cg * LANES
                for j in range(XF):
                    vec = xbuf_v[j, pl.ds(c0, LANES)]
                    xstage_v[pl.ds(j * XB + c0, LANES)] = vec
                return carry

            lax.fori_loop(0, XB // LANES, cg_body, 0)
            for j in range(XF):
                pltpu.sync_copy(
                    xstage_v.at[pl.ds(j * XB, XB)],
                    idx_hbm.at[pl.ds((f0 + j) * BATCH + b0, XB)])

        @pl.when(wid < FIELDS - X_TAIL_F)
        def _():
            pltpu.sync_copy(
                xtail_hbm.at[pl.ds(wid * BATCH, BATCH)],
                xstage_v.at[pl.ds(0, BATCH)])
            pltpu.sync_copy(
                xstage_v.at[pl.ds(0, BATCH)],
                idx_hbm.at[pl.ds((X_TAIL_F + wid) * BATCH, BATCH)])

        @pl.when(wid == 4)
        def _():
            pltpu.sync_copy(wtail_hbm, wout_v[0].at[pl.ds(0, 64 * DIM)])
            pltpu.sync_copy(
                wout_v[0].at[pl.ds(0, 64 * DIM)],
                wflat_hbm.at[pl.ds((NUM_EMB - 64) * DIM, 64 * DIM)])

        def c_of(t):
            return jnp.minimum(wid + t * NUM_WORKERS, N_WCH)

        def issue_win(c, b):
            pltpu.make_async_copy(
                wt_hbm.at[:, pl.ds(c * WCH, WCH)], win_v[b],
                win_sem[b]).start()

        def wait_win(b):
            pltpu.make_async_copy(
                wt_hbm.at[:, pl.ds(0, WCH)], win_v[b], win_sem[b]).wait()

        def issue_wout(c, b):
            pltpu.make_async_copy(
                wout_v[b], wflat_hbm.at[pl.ds(c * (WCH * DIM), WCH * DIM)],
                wout_sem[b]).start()

        def wait_wout(b):
            pltpu.make_async_copy(
                wout_v[b], wflat_hbm.at[pl.ds(0, WCH * DIM)],
                wout_sem[b]).wait()

        def w_transpose(b):
            def t_body(cg, carry):
                base = cg * (LANES * DIM)
                c0 = cg * LANES
                for d in range(DIM):
                    vec = win_v[b][d, pl.ds(c0, LANES)]
                    plsc.store_scatter(
                        wout_v[b], [base + scat_base + d], vec)
                return carry

            lax.fori_loop(0, WCH // LANES, t_body, 0)

        issue_win(c_of(0), 0)
        issue_win(c_of(1), 1)
        wait_win(0)
        w_transpose(0)
        issue_win(c_of(2), 0)
        issue_wout(c_of(0), 0)
        wait_win(1)
        w_transpose(1)
        issue_win(c_of(3), 1)
        issue_wout(c_of(1), 1)

        def w_body(tt, carry):
            for b in range(2):
                t = tt * 2 + b
                wait_win(b)
                wait_wout(b)
                w_transpose(b)
                issue_win(c_of(t + 2), b)
                issue_wout(c_of(t), b)
            return carry

        lax.fori_loop(1, N_WT // 2 - 1, w_body, 0)

        for b in range(2):
            wait_win(b)
            wait_wout(b)
            w_transpose(b)
            issue_wout(c_of(N_WT - 2 + b), b)
        for b in range(2):
            wait_wout(b)

    return prep_kernel


def _build_gather():
    mesh = plsc.VectorSubcoreMesh(core_axis_name="c", subcore_axis_name="s")

    @functools.partial(
        pl.kernel,
        mesh=mesh,
        out_type=jax.ShapeDtypeStruct((TOTAL, DIM), jnp.float32),
        scratch_types=[
            [pltpu.VMEM((CHUNK,), jnp.int32) for _ in range(NSLOT)],
            [pltpu.VMEM((CHUNK, DIM), jnp.float32) for _ in range(NSLOT)],
            [pltpu.SemaphoreType.DMA for _ in range(NSLOT)],
            [pltpu.SemaphoreType.DMA for _ in range(NSLOT)],
            [pltpu.SemaphoreType.DMA for _ in range(NSLOT)],
        ],
        compiler_params=pltpu.CompilerParams(
            use_tc_tiling_on_sc=False, needs_layout_passes=False),
    )
    def emb_kernel(idx_hbm, table_hbm, out_hbm, idx_v, rows_v,
                   idx_sem, gat_sem, out_sem):
        wid = lax.axis_index("s") * NUM_CORES + lax.axis_index("c")
        bbase = wid * CHUNK

        def issue_idx(f, b):
            pltpu.make_async_copy(
                idx_hbm.at[pl.ds(f * BATCH + bbase, CHUNK)], idx_v[b],
                idx_sem[b]).start()

        def wait_idx(b):
            pltpu.make_async_copy(
                idx_hbm.at[pl.ds(bbase, CHUNK)], idx_v[b], idx_sem[b]).wait()

        def issue_gather(b):
            pltpu.make_async_copy(
                table_hbm.at[idx_v[b]], rows_v[b], gat_sem[b]).start()

        def wait_gather(b):
            pltpu.make_async_copy(
                table_hbm.at[idx_v[b]], rows_v[b], gat_sem[b]).wait()

        def issue_out(f, b):
            pltpu.make_async_copy(
                rows_v[b], out_hbm.at[pl.ds(f * BATCH + bbase, CHUNK)],
                out_sem[b]).start()

        def wait_out(b):
            pltpu.make_async_copy(
                rows_v[b], out_hbm.at[pl.ds(bbase, CHUNK)],
                out_sem[b]).wait()

        issue_idx(0, 0)
        issue_idx(1, 1)
        wait_idx(0)
        issue_gather(0)
        wait_gather(0)
        wait_idx(1)
        issue_gather(1)
        issue_out(0, 0)
        issue_idx(2, 0)
        wait_gather(1)
        issue_out(1, 1)
        issue_idx(3, 1)

        def body(gg, carry):
            for b in range(NSLOT):
                f = gg * NSLOT + b
                wait_idx(b)
                wait_out(b)
                issue_gather(b)
                wait_gather(b)
                issue_out(f, b)
                issue_idx(f + NSLOT, b)
            return carry

        lax.fori_loop(1, FIELDS // NSLOT - 1, body, 0)

        for b in range(NSLOT):
            f = FIELDS - NSLOT + b
            wait_idx(b)
            wait_out(b)
            issue_gather(b)
            wait_gather(b)
            issue_out(f, b)
        for b in range(NSLOT):
            wait_out(b)

    return emb_kernel


TCB = 2048


def _build_tc_transpose():
    def body(in_ref, out_ref):
        out_ref[0] = in_ref[0].T

    return pl.pallas_call(
        body,
        grid=(FIELDS, BATCH // TCB),
        in_specs=[pl.BlockSpec((1, TCB, DIM), lambda f, c: (f, c, 0))],
        out_specs=pl.BlockSpec((1, DIM, TCB), lambda f, c: (f, 0, c)),
        out_shape=jax.ShapeDtypeStruct((FIELDS, DIM, BATCH), jnp.float32),
    )


_prep_kernel = _build_prep()
_emb_kernel = _build_gather()
_tc_transpose = _build_tc_transpose()


def kernel(X, W):
    Xi = X.astype(jnp.int32)
    Wt = W.T
    Xt = Xi.T
    wtail = Wt[:, NUM_EMB - 64:].T.reshape(64 * DIM)
    xtail = Xt[X_TAIL_F:].reshape((FIELDS - X_TAIL_F) * BATCH)
    w_flat, idx_flat = _prep_kernel(Wt, Xt, wtail, xtail)
    rows = _emb_kernel(idx_flat, w_flat.reshape(NUM_EMB, DIM))
    out = _tc_transpose(rows.reshape(FIELDS, BATCH, DIM))
    return out.transpose(2, 0, 1)


# speedup vs baseline: 1.3987x; 1.3987x over previous
import functools

import jax
import jax.numpy as jnp
from jax import lax
from jax.experimental import pallas as pl
from jax.experimental.pallas import tpu as pltpu
from jax.experimental.pallas import tpu_sc as plsc

NUM_EMB = 1_000_000
DIM = 32
BATCH = 16384
FIELDS = 100
TOTAL = BATCH * FIELDS

NUM_CORES = 2
NUM_SUBCORES = 16
NUM_WORKERS = NUM_CORES * NUM_SUBCORES
CHUNK = BATCH // NUM_WORKERS
NSLOT = 2
LANES = 16


def _build_gather():
    mesh = plsc.VectorSubcoreMesh(core_axis_name="c", subcore_axis_name="s")

    @functools.partial(
        pl.kernel,
        mesh=mesh,
        out_type=jax.ShapeDtypeStruct((FIELDS, DIM, BATCH), jnp.float32),
        scratch_types=[
            [pltpu.VMEM((CHUNK,), jnp.int32) for _ in range(NSLOT)],
            [pltpu.VMEM((CHUNK, DIM), jnp.float32) for _ in range(NSLOT)],
            [pltpu.VMEM((DIM, CHUNK), jnp.float32) for _ in range(NSLOT)],
            [pltpu.SemaphoreType.DMA for _ in range(NSLOT)],
            [pltpu.SemaphoreType.DMA for _ in range(NSLOT)],
            [pltpu.SemaphoreType.DMA for _ in range(NSLOT)],
        ],
        compiler_params=pltpu.CompilerParams(
            use_tc_tiling_on_sc=False, needs_layout_passes=False),
    )
    def emb_kernel(idx_hbm, table_hbm, out_hbm, idx_v, rows_v, trans_v,
                   idx_sem, gat_sem, out_sem):
        wid = lax.axis_index("s") * NUM_CORES + lax.axis_index("c")
        bbase = wid * CHUNK

        def issue_idx(f, b):
            pltpu.make_async_copy(
                idx_hbm.at[pl.ds(f * BATCH + bbase, CHUNK)], idx_v[b],
                idx_sem[b]).start()

        def wait_idx(b):
            pltpu.make_async_copy(
                idx_hbm.at[pl.ds(bbase, CHUNK)], idx_v[b], idx_sem[b]).wait()

        def issue_gather(b):
            pltpu.make_async_copy(
                table_hbm.at[idx_v[b]], rows_v[b], gat_sem[b]).start()

        def wait_gather(b):
            pltpu.make_async_copy(
                table_hbm.at[idx_v[b]], rows_v[b], gat_sem[b]).wait()

        def issue_out(f, b):
            pltpu.make_async_copy(
                trans_v[b], out_hbm.at[f, :, pl.ds(bbase, CHUNK)],
                out_sem[b]).start()

        def wait_out(b):
            pltpu.make_async_copy(
                trans_v[b], out_hbm.at[0, :, pl.ds(bbase, CHUNK)],
                out_sem[b]).wait()

        lane_iota = lax.iota(jnp.int32, LANES)

        def transpose(b):
            @plsc.parallel_loop(0, CHUNK, unroll=16)
            def j_body(j):
                col_idx = jnp.full((LANES,), j, jnp.int32)
                for dg in range(DIM // LANES):
                    vec = rows_v[b][j, pl.ds(dg * LANES, LANES)]
                    plsc.store_scatter(
                        trans_v[b], [dg * LANES + lane_iota, col_idx], vec)

        issue_idx(0, 0)
        issue_idx(1, 1)
        wait_idx(0)
        issue_gather(0)
        wait_gather(0)
        wait_idx(1)
        issue_gather(1)
        transpose(0)
        issue_out(0, 0)
        issue_idx(2, 0)
        wait_gather(1)
        wait_idx(0)
        issue_gather(0)
        transpose(1)
        issue_out(1, 1)
        issue_idx(3, 1)

        def body(gg, carry):
            for b in range(NSLOT):
                f = gg * NSLOT + b
                b2 = 1 - b
                wait_gather(b)
                wait_idx(b2)
                issue_gather(b2)
                wait_out(b)
                transpose(b)
                issue_out(f, b)
                issue_idx(f + NSLOT, b)
            return carry

        lax.fori_loop(1, FIELDS // NSLOT - 1, body, 0)

        wait_gather(0)
        wait_idx(1)
        issue_gather(1)
        wait_out(0)
        transpose(0)
        issue_out(FIELDS - 2, 0)
        wait_gather(1)
        wait_out(1)
        transpose(1)
        issue_out(FIELDS - 1, 1)
        for b in range(NSLOT):
            wait_out(b)

    return emb_kernel


_emb_kernel = _build_gather()


def kernel(X, W):
    idx = X.astype(jnp.int32).T.reshape(TOTAL)
    out = _emb_kernel(idx, W)
    return out.transpose(2, 0, 1)


# speedup vs baseline: 1.6384x; 1.1714x over previous
import functools

import jax
import jax.numpy as jnp
from jax import lax
from jax.experimental import pallas as pl
from jax.experimental.pallas import tpu as pltpu
from jax.experimental.pallas import tpu_sc as plsc

NUM_EMB = 1_000_000
DIM = 32
BATCH = 16384
FIELDS = 100
TOTAL = BATCH * FIELDS

NUM_CORES = 2
NUM_SUBCORES = 16
NUM_WORKERS = NUM_CORES * NUM_SUBCORES
CHUNK = BATCH // NUM_WORKERS
NSLOT = 2
LANES = 16


def _build_gather():
    mesh = plsc.VectorSubcoreMesh(core_axis_name="c", subcore_axis_name="s")

    @functools.partial(
        pl.kernel,
        mesh=mesh,
        out_type=jax.ShapeDtypeStruct((FIELDS * DIM * BATCH,), jnp.float32),
        scratch_types=[
            [pltpu.VMEM((CHUNK,), jnp.int32) for _ in range(NSLOT)],
            [pltpu.VMEM((CHUNK, DIM), jnp.float32) for _ in range(NSLOT)],
            [pltpu.VMEM((DIM * CHUNK,), jnp.float32) for _ in range(NSLOT)],
            [pltpu.SemaphoreType.DMA for _ in range(NSLOT)],
            [pltpu.SemaphoreType.DMA for _ in range(NSLOT)],
            [pltpu.SemaphoreType.DMA for _ in range(NSLOT)],
        ],
        compiler_params=pltpu.CompilerParams(
            use_tc_tiling_on_sc=False, needs_layout_passes=False),
    )
    def emb_kernel(idx_hbm, table_hbm, out_hbm, idx_v, rows_v, trans_v,
                   idx_sem, gat_sem, out_sem):
        wid = lax.axis_index("s") * NUM_CORES + lax.axis_index("c")
        bbase = wid * CHUNK

        def issue_idx(f, b):
            pltpu.make_async_copy(
                idx_hbm.at[pl.ds(f * BATCH + bbase, CHUNK)], idx_v[b],
                idx_sem[b]).start()

        def wait_idx(b):
            pltpu.make_async_copy(
                idx_hbm.at[pl.ds(bbase, CHUNK)], idx_v[b], idx_sem[b]).wait()

        def issue_gather(b):
            pltpu.make_async_copy(
                table_hbm.at[idx_v[b]], rows_v[b], gat_sem[b]).start()

        def wait_gather(b):
            pltpu.make_async_copy(
                table_hbm.at[idx_v[b]], rows_v[b], gat_sem[b]).wait()

        def issue_out(f, b):
            for tr in range(DIM // 8):
                pltpu.make_async_copy(
                    trans_v[b].at[pl.ds(tr * 4096, 4096)],
                    out_hbm.at[pl.ds(
                        ((f * (DIM // 8) + tr) * (BATCH // 128)
                         + wid * (CHUNK // 128)) * 1024, 4096)],
                    out_sem[b]).start()

        def wait_out(b):
            pltpu.make_async_copy(
                out_hbm.at[pl.ds(0, DIM * CHUNK)], trans_v[b],
                out_sem[b]).wait()

        lane_iota = lax.iota(jnp.int32, LANES)

        def transpose(b):
            @plsc.parallel_loop(0, CHUNK, unroll=16)
            def j_body(j):
                boff = (j // 128) * 1024 + j % 128
                col_idx = jnp.full((LANES,), boff, jnp.int32)
                for dg in range(DIM // LANES):
                    d = dg * LANES + lane_iota
                    tile_base = (d // 8) * 4096 + (d % 8) * 128
                    vec = rows_v[b][j, pl.ds(dg * LANES, LANES)]
                    plsc.store_scatter(
                        trans_v[b], [tile_base + col_idx], vec)

        issue_idx(0, 0)
        issue_idx(1, 1)
        wait_idx(0)
        issue_gather(0)
        wait_gather(0)
        wait_idx(1)
        issue_gather(1)
        transpose(0)
        issue_out(0, 0)
        issue_idx(2, 0)
        wait_gather(1)
        wait_idx(0)
        issue_gather(0)
        transpose(1)
        issue_out(1, 1)
        issue_idx(3, 1)

        def body(gg, carry):
            for b in range(NSLOT):
                f = gg * NSLOT + b
                b2 = 1 - b
                wait_gather(b)
                wait_idx(b2)
                issue_gather(b2)
                wait_out(b)
                transpose(b)
                issue_out(f, b)
                issue_idx(f + NSLOT, b)
            return carry

        lax.fori_loop(1, FIELDS // NSLOT - 1, body, 0)

        wait_gather(0)
        wait_idx(1)
        issue_gather(1)
        wait_out(0)
        transpose(0)
        issue_out(FIELDS - 2, 0)
        wait_gather(1)
        wait_out(1)
        transpose(1)
        issue_out(FIELDS - 1, 1)
        for b in range(NSLOT):
            wait_out(b)

    return emb_kernel


_emb_kernel = _build_gather()


def kernel(X, W):
    idx = X.astype(jnp.int32).T.reshape(TOTAL)
    out = _emb_kernel(idx, W)
    o5 = out.reshape(FIELDS, DIM // 8, BATCH // 128, 8, 128)
    return o5.transpose(2, 4, 0, 1, 3).reshape(BATCH, FIELDS, DIM)
